# baseline (speedup 1.0000x reference)
import functools

import jax
import jax.numpy as jnp
from jax import lax
from jax.experimental import pallas as pl
from jax.experimental.pallas import tpu as pltpu
from jax.experimental.pallas import tpu_sc as plsc

EMBED_D = 64
GROUP = 128
GPC = 4
CHUNK = GROUP * GPC


@functools.lru_cache(maxsize=None)
def _emb_gather(num_idx: int, vocab: int):
    info = plsc.get_sparse_core_info()
    nc, ns = info.num_cores, info.num_subcores
    nw = nc * ns
    rows_total = num_idx // GROUP
    rpw = rows_total // nw
    bpw = num_idx // nw
    nchunks = bpw // CHUNK
    vpw = vocab // nw

    mesh = plsc.VectorSubcoreMesh(core_axis_name="c", subcore_axis_name="s")

    @functools.partial(
        pl.kernel,
        mesh=mesh,
        out_type=(
            jax.ShapeDtypeStruct((num_idx, EMBED_D), jnp.float32),
            jax.ShapeDtypeStruct((vocab, EMBED_D), jnp.float32),
        ),
        scratch_types=[
            pltpu.VMEM((rpw, GROUP), jnp.int32),
            pltpu.VMEM((2, CHUNK, EMBED_D), jnp.float32),
            pltpu.SemaphoreType.DMA((2,)),
            pltpu.SemaphoreType.DMA((2,)),
            pltpu.SemaphoreType.DMA,
        ],
        compiler_params=pltpu.CompilerParams(use_tc_tiling_on_sc=False),
    )
    def k(idx_hbm, table_hbm, out_hbm, tcopy_hbm, idx_v, rows_v,
          gsem, wsem, csem):
        wid = lax.axis_index("s") * nc + lax.axis_index("c")
        row0 = wid * rpw
        base = wid * bpw
        pltpu.async_copy(table_hbm.at[pl.ds(wid * vpw, vpw)],
                         tcopy_hbm.at[pl.ds(wid * vpw, vpw)], csem)
        pltpu.sync_copy(idx_hbm.at[pl.ds(row0, rpw)], idx_v)

        def fire_gathers(g, b):
            for j in range(GPC):
                pltpu.async_copy(
                    table_hbm.at[idx_v.at[g * GPC + j]],
                    rows_v.at[b, pl.ds(j * GROUP, GROUP)],
                    gsem.at[b],
                )

        def drain_gathers(g, b):
            for j in range(GPC):
                pltpu.make_async_copy(
                    table_hbm.at[idx_v.at[g * GPC + j]],
                    rows_v.at[b, pl.ds(j * GROUP, GROUP)],
                    gsem.at[b],
                ).wait()

        fire_gathers(0, 0)

        def body(g, carry):
            b = lax.rem(g, 2)
            nb = 1 - b
            drain_gathers(g, b)

            @pl.when(g >= 1)
            def _():
                pltpu.make_async_copy(
                    rows_v.at[nb],
                    out_hbm.at[pl.ds(base + (g - 1) * CHUNK, CHUNK)],
                    wsem.at[nb],
                ).wait()

            @pl.when(g + 1 < nchunks)
            def _():
                fire_gathers(g + 1, nb)

            pltpu.async_copy(
                rows_v.at[b],
                out_hbm.at[pl.ds(base + g * CHUNK, CHUNK)],
                wsem.at[b],
            )
            return carry

        lax.fori_loop(0, nchunks, body, 0)
        lastb = (nchunks - 1) % 2
        pltpu.make_async_copy(
            rows_v.at[lastb],
            out_hbm.at[pl.ds(base + (nchunks - 1) * CHUNK, CHUNK)],
            wsem.at[lastb],
        ).wait()
        pltpu.make_async_copy(table_hbm.at[pl.ds(wid * vpw, vpw)],
                              tcopy_hbm.at[pl.ds(wid * vpw, vpw)], csem).wait()

    return k


def kernel(inputs, token_embed_weights):
    idx = inputs.astype(jnp.int32)
    num_idx = idx.size
    vocab = token_embed_weights.shape[0]
    idx2 = idx.reshape(num_idx // GROUP, GROUP)
    out, tcopy = _emb_gather(num_idx, vocab)(idx2, token_embed_weights)
    return out.reshape(inputs.shape + (EMBED_D,)), tcopy


# speedup vs baseline: 4.1827x; 4.1827x over previous
import functools

import jax
import jax.numpy as jnp
from jax import lax
from jax.experimental import pallas as pl
from jax.experimental.pallas import tpu as pltpu
from jax.experimental.pallas import tpu_sc as plsc

EMBED_D = 64
GROUP = 128
GPC = 4
CHUNK = GROUP * GPC


@functools.lru_cache(maxsize=None)
def _emb_gather(num_idx: int):
    info = plsc.get_sparse_core_info()
    nc, ns = info.num_cores, info.num_subcores
    nw = nc * ns
    rows_total = num_idx // GROUP
    rpw = rows_total // nw
    bpw = num_idx // nw
    nchunks = bpw // CHUNK

    mesh = plsc.VectorSubcoreMesh(core_axis_name="c", subcore_axis_name="s")

    @functools.partial(
        pl.kernel,
        mesh=mesh,
        out_type=jax.ShapeDtypeStruct((num_idx, EMBED_D), jnp.float32),
        scratch_types=[
            pltpu.VMEM((rpw, GROUP), jnp.int32),
            pltpu.VMEM((2, CHUNK, EMBED_D), jnp.float32),
            pltpu.SemaphoreType.DMA((2,)),
            pltpu.SemaphoreType.DMA((2,)),
        ],
        compiler_params=pltpu.CompilerParams(use_tc_tiling_on_sc=False),
    )
    def k(idx_hbm, table_hbm, out_hbm, idx_v, rows_v, gsem, wsem):
        wid = lax.axis_index("s") * nc + lax.axis_index("c")
        row0 = wid * rpw
        base = wid * bpw
        pltpu.sync_copy(idx_hbm.at[pl.ds(row0, rpw)], idx_v)

        def fire_gathers(g, b):
            for j in range(GPC):
                pltpu.async_copy(
                    table_hbm.at[idx_v.at[g * GPC + j]],
                    rows_v.at[b, pl.ds(j * GROUP, GROUP)],
                    gsem.at[b],
                )

        def drain_gathers(g, b):
            for j in range(GPC):
                pltpu.make_async_copy(
                    table_hbm.at[idx_v.at[g * GPC + j]],
                    rows_v.at[b, pl.ds(j * GROUP, GROUP)],
                    gsem.at[b],
                ).wait()

        fire_gathers(0, 0)

        def body(g, carry):
            b = lax.rem(g, 2)
            nb = 1 - b
            drain_gathers(g, b)

            @pl.when(g >= 1)
            def _():
                pltpu.make_async_copy(
                    rows_v.at[nb],
                    out_hbm.at[pl.ds(base + (g - 1) * CHUNK, CHUNK)],
                    wsem.at[nb],
                ).wait()

            @pl.when(g + 1 < nchunks)
            def _():
                fire_gathers(g + 1, nb)

            pltpu.async_copy(
                rows_v.at[b],
                out_hbm.at[pl.ds(base + g * CHUNK, CHUNK)],
                wsem.at[b],
            )
            return carry

        lax.fori_loop(0, nchunks, body, 0)
        lastb = (nchunks - 1) % 2
        pltpu.make_async_copy(
            rows_v.at[lastb],
            out_hbm.at[pl.ds(base + (nchunks - 1) * CHUNK, CHUNK)],
            wsem.at[lastb],
        ).wait()

    return k


def _tc_copy_body(t_ref, o_ref):
    o_ref[...] = t_ref[...]


@functools.lru_cache(maxsize=None)
def _tc_copy(rows: int):
    blk = 10000
    return pl.pallas_call(
        _tc_copy_body,
        grid=(rows // blk,),
        in_specs=[pl.BlockSpec((blk, 128), lambda i: (i, 0))],
        out_specs=pl.BlockSpec((blk, 128), lambda i: (i, 0)),
        out_shape=jax.ShapeDtypeStruct((rows, 128), jnp.float32),
    )


def kernel(inputs, token_embed_weights):
    idx = inputs.astype(jnp.int32)
    num_idx = idx.size
    vocab = token_embed_weights.shape[0]
    idx2 = idx.reshape(num_idx // GROUP, GROUP)
    out = _emb_gather(num_idx)(idx2, token_embed_weights)
    tab2 = token_embed_weights.reshape(vocab // 2, 2 * EMBED_D)
    tcopy = _tc_copy(vocab // 2)(tab2).reshape(vocab, EMBED_D)
    return out.reshape(inputs.shape + (EMBED_D,)), tcopy


# speedup vs baseline: 7.9782x; 1.9074x over previous
import functools

import jax
import jax.numpy as jnp
from jax import lax
from jax.experimental import pallas as pl
from jax.experimental.pallas import tpu as pltpu
from jax.experimental.pallas import tpu_sc as plsc

EMBED_D = 64
LANES = 128
RELAYOUT_BLK = 4096


def _relayout_body(t_ref, o_ref):
    o_ref[:, pl.ds(0, EMBED_D)] = t_ref[...].T


@functools.lru_cache(maxsize=None)
def _tc_relayout(vocab: int):
    nblk = (vocab + RELAYOUT_BLK - 1) // RELAYOUT_BLK
    return pl.pallas_call(
        _relayout_body,
        grid=(nblk,),
        in_specs=[pl.BlockSpec((EMBED_D, RELAYOUT_BLK), lambda i: (0, i))],
        out_specs=pl.BlockSpec((RELAYOUT_BLK, LANES), lambda i: (i, 0)),
        out_shape=jax.ShapeDtypeStruct((vocab, LANES), jnp.float32),
    )


@functools.lru_cache(maxsize=None)
def _sc_gather(n_t: int, n_b: int, vocab: int):
    info = plsc.get_sparse_core_info()
    nc, ns = info.num_cores, info.num_subcores
    nw = nc * ns
    tiles_b = n_b // LANES
    tiles_t = n_t // 8
    tpw = (tiles_t * tiles_b) // nw

    mesh = plsc.VectorSubcoreMesh(core_axis_name="c", subcore_axis_name="s")

    @functools.partial(
        pl.kernel,
        mesh=mesh,
        out_type=jax.ShapeDtypeStruct((n_t, n_b, EMBED_D), jnp.float32),
        scratch_types=[
            pltpu.VMEM((8, LANES), jnp.int32),
            pltpu.VMEM((LANES, LANES), jnp.float32),
            pltpu.VMEM((LANES, EMBED_D), jnp.float32),
            pltpu.SemaphoreType.DMA,
        ],
        compiler_params=pltpu.CompilerParams(use_tc_tiling_on_sc=True),
    )
    def k(idx_hbm, r5_hbm, out_hbm, idx_v, gbuf, cbuf, sem):
        wid = lax.axis_index("s") * nc + lax.axis_index("c")

        def tile_body(j, carry):
            tile = wid * tpw + j
            t_hi = tile // tiles_b
            b_hi = tile % tiles_b
            pltpu.sync_copy(
                idx_hbm.at[pl.ds(t_hi * 8, 8), pl.ds(b_hi * LANES, LANES)],
                idx_v)

            def t_body(t_lo, carry2):
                pltpu.async_copy(r5_hbm.at[idx_v.at[t_lo]], gbuf, sem).wait()
                for r in range(LANES):
                    for c in range(0, EMBED_D, 16):
                        cbuf[r, pl.ds(c, 16)] = gbuf[r, pl.ds(c, 16)]
                pltpu.sync_copy(
                    cbuf,
                    out_hbm.at[t_hi * 8 + t_lo,
                               pl.ds(b_hi * LANES, LANES), :])
                return carry2

            lax.fori_loop(0, 8, t_body, 0)
            return carry

        lax.fori_loop(0, tpw, tile_body, 0)

    return k


def kernel(inputs, token_embed_weights):
    idx_t = inputs.astype(jnp.int32).T
    tab_t = token_embed_weights.T
    vocab = token_embed_weights.shape[0]
    r5 = _tc_relayout(vocab)(tab_t)
    n_t, n_b = idx_t.shape
    out4 = _sc_gather(n_t, n_b, vocab)(idx_t, r5)
    out = jnp.transpose(out4, (1, 0, 2))
    return out, token_embed_weights


# speedup vs baseline: 11.8646x; 1.4871x over previous
import functools

import jax
import jax.numpy as jnp
from jax import lax
from jax.experimental import pallas as pl
from jax.experimental.pallas import tpu as pltpu
from jax.experimental.pallas import tpu_sc as plsc

EMBED_D = 64
LANES = 128
RELAYOUT_BLK = 8192


def _relayout_body(t_ref, o_ref, o2_ref):
    x = t_ref[...]
    o_ref[:, pl.ds(0, EMBED_D)] = x.T
    o2_ref[...] = x


@functools.lru_cache(maxsize=None)
def _tc_relayout(vocab: int):
    nblk = (vocab + RELAYOUT_BLK - 1) // RELAYOUT_BLK
    return pl.pallas_call(
        _relayout_body,
        grid=(nblk,),
        in_specs=[pl.BlockSpec((EMBED_D, RELAYOUT_BLK), lambda i: (0, i))],
        out_specs=[
            pl.BlockSpec((RELAYOUT_BLK, LANES), lambda i: (i, 0)),
            pl.BlockSpec((EMBED_D, RELAYOUT_BLK), lambda i: (0, i)),
        ],
        out_shape=[
            jax.ShapeDtypeStruct((vocab, LANES), jnp.float32),
            jax.ShapeDtypeStruct((EMBED_D, vocab), jnp.float32),
        ],
    )


@functools.lru_cache(maxsize=None)
def _sc_gather(n_t: int, n_b: int, vocab: int):
    info = plsc.get_sparse_core_info()
    nc, ns = info.num_cores, info.num_subcores
    nw = nc * ns
    tiles_b = n_b // LANES
    tiles_t = n_t // 8
    tpw = (tiles_t * tiles_b) // nw
    bpw = tpw * 8

    mesh = plsc.VectorSubcoreMesh(core_axis_name="c", subcore_axis_name="s")

    @functools.partial(
        pl.kernel,
        mesh=mesh,
        out_type=jax.ShapeDtypeStruct((n_t, n_b, EMBED_D), jnp.float32),
        scratch_types=[
            pltpu.VMEM((2, 8, LANES), jnp.int32),
            pltpu.VMEM((2, LANES, LANES), jnp.float32),
            pltpu.VMEM((2, LANES, EMBED_D), jnp.float32),
            pltpu.SemaphoreType.DMA((2,)),
            pltpu.SemaphoreType.DMA((2,)),
        ],
        compiler_params=pltpu.CompilerParams(use_tc_tiling_on_sc=True),
    )
    def k(idx_hbm, r5_hbm, out_hbm, idx_v, gbuf, cbuf, gsem, wsem):
        wid = lax.axis_index("s") * nc + lax.axis_index("c")

        def fire(kk):
            j = kk // 8
            t_lo = lax.rem(kk, 8)
            jb = lax.rem(j, 2)
            tile = wid * tpw + j

            @pl.when(t_lo == 0)
            def _():
                pltpu.sync_copy(
                    idx_hbm.at[pl.ds((tile // tiles_b) * 8, 8),
                               pl.ds(lax.rem(tile, tiles_b) * LANES, LANES)],
                    idx_v.at[jb])

            pltpu.async_copy(r5_hbm.at[idx_v.at[jb, t_lo]],
                             gbuf.at[lax.rem(kk, 2)],
                             gsem.at[lax.rem(kk, 2)])

        fire(0)

        def body(kk, carry):
            b = lax.rem(kk, 2)
            j = kk // 8
            t_lo = lax.rem(kk, 8)
            tile = wid * tpw + j

            @pl.when(kk + 1 < bpw)
            def _():
                fire(kk + 1)

            pltpu.make_async_copy(r5_hbm.at[idx_v.at[lax.rem(j, 2), t_lo]],
                                  gbuf.at[b], gsem.at[b]).wait()

            @pl.when(kk >= 2)
            def _():
                _wait_write(kk - 2)

            for r in range(LANES):
                for c in range(0, EMBED_D, 16):
                    cbuf[b, r, pl.ds(c, 16)] = gbuf[b, r, pl.ds(c, 16)]

            pltpu.async_copy(
                cbuf.at[b],
                out_hbm.at[(tile // tiles_b) * 8 + t_lo,
                           pl.ds(lax.rem(tile, tiles_b) * LANES, LANES), :],
                wsem.at[b])
            return carry

        def _wait_write(kk):
            b = lax.rem(kk, 2)
            j = kk // 8
            t_lo = lax.rem(kk, 8)
            tile = wid * tpw + j
            pltpu.make_async_copy(
                cbuf.at[b],
                out_hbm.at[(tile // tiles_b) * 8 + t_lo,
                           pl.ds(lax.rem(tile, tiles_b) * LANES, LANES), :],
                wsem.at[b]).wait()

        lax.fori_loop(0, bpw, body, 0)
        _wait_write(bpw - 2)
        _wait_write(bpw - 1)

    return k


def kernel(inputs, token_embed_weights):
    idx_t = inputs.astype(jnp.int32).T
    tab_t = token_embed_weights.T
    vocab = token_embed_weights.shape[0]
    r5, tcopy_t = _tc_relayout(vocab)(tab_t)
    n_t, n_b = idx_t.shape
    out4 = _sc_gather(n_t, n_b, vocab)(idx_t, r5)
    out = jnp.transpose(out4, (1, 0, 2))
    return out, tcopy_t.T
